# baseline (device time: 64609 ns/iter reference)
import functools

import jax
import jax.numpy as jnp
from jax import lax
from jax.experimental import pallas as pl
from jax.experimental.pallas import tpu as pltpu

N_DEV = 4
SQ = 256
SKV = 4096
D = 1024
DH = 128
H_LOCAL = 8
SCALE = 0.08838834764831843


def kernel(x, Wq, Wo, K_ext, V_ext):
    xr = x.reshape(SQ, D)
    Kr = K_ext.reshape(SKV, 8 * DH)
    Vr = V_ext.reshape(SKV, 8 * DH)

    def body(x_hbm, wq_hbm, wo_hbm, k_hbm, v_hbm, out_ref,
             x_v, wq_v, wo_v, k_v, v_v, send_buf, comm,
             load_sems, send_sems, recv_sems):
        my_pos = lax.axis_index("i")

        cp_x = pltpu.make_async_copy(x_hbm, x_v, load_sems.at[0])
        cp_wq = pltpu.make_async_copy(wq_hbm, wq_v, load_sems.at[1])
        cp_k = pltpu.make_async_copy(
            k_hbm.at[:, pl.ds(my_pos * 2 * DH, 2 * DH)], k_v, load_sems.at[2])
        cp_v = pltpu.make_async_copy(
            v_hbm.at[:, pl.ds(my_pos * 2 * DH, 2 * DH)], v_v, load_sems.at[3])
        cp_wo = pltpu.make_async_copy(wo_hbm, wo_v, load_sems.at[4])
        cp_x.start()
        cp_wq.start()
        cp_k.start()
        cp_v.start()
        cp_wo.start()

        barrier = pltpu.get_barrier_semaphore()
        for d in range(1, N_DEV):
            pl.semaphore_signal(
                barrier, inc=1,
                device_id=((my_pos + d) % N_DEV,),
                device_id_type=pl.DeviceIdType.MESH,
            )
        pl.semaphore_wait(barrier, N_DEV - 1)

        cp_x.wait()
        cp_wq.wait()
        q = jnp.dot(x_v[...].astype(jnp.bfloat16),
                    wq_v[...].astype(jnp.bfloat16),
                    preferred_element_type=jnp.float32)
        q = (q * SCALE).astype(jnp.bfloat16)

        cp_k.wait()
        kb = k_v[...].astype(jnp.bfloat16)
        cp_v.wait()
        vb = v_v[...].astype(jnp.bfloat16)

        outs = []
        for h in range(H_LOCAL):
            qh = q[:, h * DH:(h + 1) * DH]
            kh = kb[:, (h // 4) * DH:(h // 4 + 1) * DH]
            vh = vb[:, (h // 4) * DH:(h // 4 + 1) * DH]
            s = lax.dot_general(
                qh, kh, (((1,), (1,)), ((), ())),
                preferred_element_type=jnp.float32)
            m = jnp.max(s, axis=1, keepdims=True)
            p = jnp.exp(s - m)
            l = jnp.sum(p, axis=1, keepdims=True)
            o = jnp.dot(p.astype(jnp.bfloat16), vh,
                        preferred_element_type=jnp.float32) / l
            outs.append(o.astype(jnp.bfloat16))
        attn = jnp.concatenate(outs, axis=1)

        cp_wo.wait()
        partial = jnp.dot(attn, wo_v[...].astype(jnp.bfloat16),
                          preferred_element_type=jnp.float32)
        send_buf[...] = partial.astype(jnp.bfloat16)

        rdmas = []
        for d in range(1, N_DEV):
            r = pltpu.make_async_remote_copy(
                src_ref=send_buf,
                dst_ref=comm.at[d - 1],
                send_sem=send_sems.at[d - 1],
                recv_sem=recv_sems.at[d - 1],
                device_id=((my_pos + d) % N_DEV,),
                device_id_type=pl.DeviceIdType.MESH,
            )
            r.start()
            rdmas.append(r)

        acc = partial
        for d in range(1, N_DEV):
            rdmas[d - 1].wait_recv()
            acc = acc + comm[d - 1].astype(jnp.float32)
        out_ref[...] = acc
        for r in rdmas:
            r.wait_send()

        @functools.partial(pl.run_scoped, exit_sem=pltpu.SemaphoreType.REGULAR)
        def _(exit_sem):
            for d in range(1, N_DEV):
                pl.semaphore_signal(
                    exit_sem, inc=1,
                    device_id=((my_pos + d) % N_DEV,),
                    device_id_type=pl.DeviceIdType.MESH,
                )
            pl.semaphore_wait(exit_sem, N_DEV - 1)

    out = pl.pallas_call(
        body,
        out_shape=jax.ShapeDtypeStruct((SQ, D), jnp.float32),
        in_specs=[pl.BlockSpec(memory_space=pl.ANY)] * 5,
        out_specs=pl.BlockSpec(memory_space=pltpu.VMEM),
        scratch_shapes=[
            pltpu.VMEM((SQ, D), jnp.float32),
            pltpu.VMEM((D, D), jnp.float32),
            pltpu.VMEM((D, D), jnp.float32),
            pltpu.VMEM((SKV, 2 * DH), jnp.float32),
            pltpu.VMEM((SKV, 2 * DH), jnp.float32),
            pltpu.VMEM((SQ, D), jnp.bfloat16),
            pltpu.VMEM((N_DEV - 1, SQ, D), jnp.bfloat16),
            pltpu.SemaphoreType.DMA((5,)),
            pltpu.SemaphoreType.DMA((N_DEV - 1,)),
            pltpu.SemaphoreType.DMA((N_DEV - 1,)),
        ],
        compiler_params=pltpu.CompilerParams(
            collective_id=0, vmem_limit_bytes=64 * 1024 * 1024),
    )(xr, Wq, Wo, Kr, Vr)
    return out[None]


# device time: 38995 ns/iter; 1.6569x vs baseline; 1.6569x over previous
import functools

import jax
import jax.numpy as jnp
from jax import lax
from jax.experimental import pallas as pl
from jax.experimental.pallas import tpu as pltpu

N_DEV = 4
SQ = 256
SKV = 4096
D = 1024
DH = 128
H_LOCAL = 8
SCALE = 0.08838834764831843


def kernel(x, Wq, Wo, K_ext, V_ext):
    def body(x_hbm, wq_hbm, wo_hbm, k_hbm, v_hbm, out_ref,
             x_v, wq_v, wo_v, k_v, v_v, send_buf, comm,
             load_sems, send_sems, recv_sems):
        my_pos = lax.axis_index("i")

        cp_x = pltpu.make_async_copy(x_hbm.at[0], x_v, load_sems.at[0])
        cp_wq = pltpu.make_async_copy(wq_hbm, wq_v, load_sems.at[1])
        cp_wo = pltpu.make_async_copy(wo_hbm, wo_v, load_sems.at[2])
        cp_x.start()
        cp_wq.start()
        kv_cps = []
        for s in range(2):
            ck = pltpu.make_async_copy(
                k_hbm.at[0, :, 2 * my_pos + s, :], k_v.at[s],
                load_sems.at[3 + s])
            cv = pltpu.make_async_copy(
                v_hbm.at[0, :, 2 * my_pos + s, :], v_v.at[s],
                load_sems.at[5 + s])
            ck.start()
            cv.start()
            kv_cps.append((ck, cv))
        cp_wo.start()

        barrier = pltpu.get_barrier_semaphore()
        for d in range(1, N_DEV):
            pl.semaphore_signal(
                barrier, inc=1,
                device_id=((my_pos + d) % N_DEV,),
                device_id_type=pl.DeviceIdType.MESH,
            )
        pl.semaphore_wait(barrier, N_DEV - 1)

        cp_x.wait()
        cp_wq.wait()
        q = jnp.dot(x_v[...].astype(jnp.bfloat16),
                    wq_v[...].astype(jnp.bfloat16),
                    preferred_element_type=jnp.float32)
        q = (q * SCALE).astype(jnp.bfloat16)

        kbs, vbs = [], []
        for s in range(2):
            ck, cv = kv_cps[s]
            ck.wait()
            kbs.append(k_v[s].astype(jnp.bfloat16))
            cv.wait()
            vbs.append(v_v[s].astype(jnp.bfloat16))

        outs = []
        for h in range(H_LOCAL):
            qh = q[:, h * DH:(h + 1) * DH]
            kh = kbs[h // 4]
            vh = vbs[h // 4]
            s = lax.dot_general(
                qh, kh, (((1,), (1,)), ((), ())),
                preferred_element_type=jnp.float32)
            m = jnp.max(s, axis=1, keepdims=True)
            p = jnp.exp(s - m)
            l = jnp.sum(p, axis=1, keepdims=True)
            o = jnp.dot(p.astype(jnp.bfloat16), vh,
                        preferred_element_type=jnp.float32) / l
            outs.append(o.astype(jnp.bfloat16))
        attn = jnp.concatenate(outs, axis=1)

        cp_wo.wait()
        partial = jnp.dot(attn, wo_v[...].astype(jnp.bfloat16),
                          preferred_element_type=jnp.float32)
        send_buf[...] = partial.astype(jnp.bfloat16)

        rdmas = []
        for d in range(1, N_DEV):
            r = pltpu.make_async_remote_copy(
                src_ref=send_buf,
                dst_ref=comm.at[d - 1],
                send_sem=send_sems.at[d - 1],
                recv_sem=recv_sems.at[d - 1],
                device_id=((my_pos + d) % N_DEV,),
                device_id_type=pl.DeviceIdType.MESH,
            )
            r.start()
            rdmas.append(r)

        acc = partial
        for d in range(1, N_DEV):
            rdmas[d - 1].wait_recv()
            acc = acc + comm[d - 1].astype(jnp.float32)
        out_ref[0] = acc
        for r in rdmas:
            r.wait_send()

        @functools.partial(pl.run_scoped, exit_sem=pltpu.SemaphoreType.REGULAR)
        def _(exit_sem):
            for d in range(1, N_DEV):
                pl.semaphore_signal(
                    exit_sem, inc=1,
                    device_id=((my_pos + d) % N_DEV,),
                    device_id_type=pl.DeviceIdType.MESH,
                )
            pl.semaphore_wait(exit_sem, N_DEV - 1)

    return pl.pallas_call(
        body,
        out_shape=jax.ShapeDtypeStruct((1, SQ, D), jnp.float32),
        in_specs=[pl.BlockSpec(memory_space=pl.ANY)] * 5,
        out_specs=pl.BlockSpec(memory_space=pltpu.VMEM),
        scratch_shapes=[
            pltpu.VMEM((SQ, D), jnp.float32),
            pltpu.VMEM((D, D), jnp.float32),
            pltpu.VMEM((D, D), jnp.float32),
            pltpu.VMEM((2, SKV, DH), jnp.float32),
            pltpu.VMEM((2, SKV, DH), jnp.float32),
            pltpu.VMEM((SQ, D), jnp.bfloat16),
            pltpu.VMEM((N_DEV - 1, SQ, D), jnp.bfloat16),
            pltpu.SemaphoreType.DMA((7,)),
            pltpu.SemaphoreType.DMA((N_DEV - 1,)),
            pltpu.SemaphoreType.DMA((N_DEV - 1,)),
        ],
        compiler_params=pltpu.CompilerParams(
            collective_id=0, vmem_limit_bytes=64 * 1024 * 1024),
    )(x, Wq, Wo, K_ext, V_ext)
